# Initial kernel scaffold; baseline (speedup 1.0000x reference)
import jax
import jax.numpy as jnp
from jax.experimental import pallas as pl


def kernel(x, router_w, w1, w2):
    raise NotImplementedError("write your pallas kernel here")


# speedup vs baseline: 3.8059x; 3.8059x over previous
import functools
import math

import jax
import jax.numpy as jnp
from jax.experimental import pallas as pl
from jax.experimental.pallas import tpu as pltpu

D_MODEL = 768
D_FF = 3072
N_EXP = 16
T = 2048


def _router_body(x_ref, rw_ref, gate_ref, aux_ref):
    x = x_ref[...]
    rw = rw_ref[...]
    logits = jax.lax.dot_general(
        x, rw, (((1,), (1,)), ((), ())), preferred_element_type=jnp.float32
    )
    m = jnp.max(logits, axis=1, keepdims=True)
    e = jnp.exp(logits - m)
    probs = e / jnp.sum(e, axis=1, keepdims=True)
    lane = jax.lax.broadcasted_iota(jnp.int32, probs.shape, 1)
    m1 = jnp.max(probs, axis=1, keepdims=True)
    i1 = jnp.min(jnp.where(probs >= m1, lane, N_EXP), axis=1, keepdims=True)
    mask1 = lane == i1
    probs2 = jnp.where(mask1, -1.0, probs)
    m2 = jnp.max(probs2, axis=1, keepdims=True)
    i2 = jnp.min(jnp.where(probs2 >= m2, lane, N_EXP), axis=1, keepdims=True)
    denom = m1 + m2
    gate = jnp.where(mask1, m1 / denom, 0.0) + jnp.where(lane == i2, m2 / denom, 0.0)
    gate_ref[...] = gate
    usage = jnp.sum(probs, axis=0, keepdims=True) * (1.0 / T)
    aux = N_EXP * jnp.sum(usage * usage)
    aux_ref[0, 0] = aux


def _ffn_body(x_ref, gate_ref, w1_ref, w2_ref, out_ref):
    e = pl.program_id(0)
    f = pl.program_id(1)
    x = x_ref[...]
    h = jax.lax.dot_general(
        x, w1_ref[0], (((1,), (1,)), ((), ())), preferred_element_type=jnp.float32
    )
    h = 0.5 * h * (1.0 + jax.lax.erf(h * (1.0 / math.sqrt(2.0))))
    y = jax.lax.dot_general(
        h, w2_ref[0], (((1,), (1,)), ((), ())), preferred_element_type=jnp.float32
    )
    lane = jax.lax.broadcasted_iota(jnp.int32, gate_ref.shape, 1)
    gcol = jnp.sum(jnp.where(lane == e, gate_ref[...], 0.0), axis=1, keepdims=True)

    @pl.when(jnp.logical_and(e == 0, f == 0))
    def _():
        out_ref[...] = jnp.zeros_like(out_ref)

    out_ref[...] += gcol * y


def kernel(x, router_w, w1, w2):
    B, S, D = x.shape
    x_flat = x.reshape(-1, D)

    gate, aux = pl.pallas_call(
        _router_body,
        out_shape=(
            jax.ShapeDtypeStruct((T, N_EXP), jnp.float32),
            jax.ShapeDtypeStruct((1, 1), jnp.float32),
        ),
        in_specs=[
            pl.BlockSpec((T, D_MODEL), lambda: (0, 0)),
            pl.BlockSpec((N_EXP, D_MODEL), lambda: (0, 0)),
        ],
        out_specs=(
            pl.BlockSpec((T, N_EXP), lambda: (0, 0)),
            pl.BlockSpec(memory_space=pltpu.SMEM),
        ),
    )(x_flat, router_w)

    FB = 768
    out = pl.pallas_call(
        _ffn_body,
        grid=(N_EXP, D_FF // FB),
        out_shape=jax.ShapeDtypeStruct((T, D_MODEL), jnp.float32),
        in_specs=[
            pl.BlockSpec((T, D_MODEL), lambda e, f: (0, 0)),
            pl.BlockSpec((T, N_EXP), lambda e, f: (0, 0)),
            pl.BlockSpec((1, FB, D_MODEL), lambda e, f: (e, f, 0)),
            pl.BlockSpec((1, D_MODEL, FB), lambda e, f: (e, 0, f)),
        ],
        out_specs=pl.BlockSpec((T, D_MODEL), lambda e, f: (0, 0)),
    )(x_flat, gate, w1, w2)

    return out.reshape(B, S, D), aux[0, 0]
